# Initial kernel scaffold; baseline (speedup 1.0000x reference)
import jax
import jax.numpy as jnp
from jax.experimental import pallas as pl


def kernel(x, params):
    raise NotImplementedError("write your pallas kernel here")


# speedup vs baseline: 6.8638x; 6.8638x over previous
import functools

import jax
import jax.numpy as jnp
from jax.experimental import pallas as pl
from jax.experimental.pallas import tpu as pltpu

_HIGH = jax.lax.Precision.HIGHEST
_K = 16


def _dot(a, b):
    return jnp.dot(a, b)


def _relu(v):
    return jnp.maximum(v, 0.0)


def _sqdist(a, b):
    d = -2.0 * jax.lax.dot_general(a, b, (((1,), (1,)), ((), ())))
    d = d + jnp.sum(a * a, -1)[:, None]
    d = d + jnp.sum(b * b, -1)[None, :]
    return d


def _mlp_first_kernel(x_ref, w0, b0, w1, b1, o_ref):
    x = x_ref[0]
    h = _relu(_dot(x, w0[...]) + b0[...])
    o_ref[0] = _dot(h, w1[...]) + b1[...]


def _transformer_kernel(keff, xyz_ref, f_ref,
                        fc1w, fc1b, fc2w, fc2b,
                        qw, qb, kw, kb, vw, vb,
                        d1w, d1b, d2w, d2b,
                        g1w, g1b, g2w, g2b,
                        o_ref, logit_s, wsrc_s):
    xyz = xyz_ref[0]
    feats = f_ref[0]
    n = xyz.shape[0]
    dm = qw.shape[1]

    dists = _sqdist(xyz, xyz)
    h = _dot(feats, fc1w[...]) + fc1b[...]
    q = _dot(h, qw[...]) + qb[...]
    kf = _dot(h, kw[...]) + kb[...]
    vf = _dot(h, vw[...]) + vb[...]
    src = jnp.concatenate([kf, vf, xyz], axis=1)

    col = jax.lax.broadcasted_iota(jnp.int32, (n, n), 1)
    scale = 1.0 / (dm ** 0.5)
    for k in range(keff):
        idx = jnp.argmin(dists, axis=-1)
        sel = col == idx[:, None]
        dists = jnp.where(sel, jnp.inf, dists)
        g = jnp.dot(sel.astype(jnp.float32), src, precision=_HIGH)
        kf_k = g[:, 0:dm]
        vf_k = g[:, dm:2 * dm]
        nxyz = g[:, 2 * dm:2 * dm + 3]
        pos = xyz - nxyz
        pe = _dot(_relu(_dot(pos, d1w[...]) + d1b[...]), d2w[...]) + d2b[...]
        lg = _dot(_relu(_dot(q - kf_k + pe, g1w[...]) + g1b[...]),
                  g2w[...]) + g2b[...]
        logit_s[:, k * dm:(k + 1) * dm] = lg * scale
        wsrc_s[:, k * dm:(k + 1) * dm] = vf_k + pe

    m = logit_s[:, 0:dm]
    for k in range(1, keff):
        m = jnp.maximum(m, logit_s[:, k * dm:(k + 1) * dm])
    ssum = jnp.zeros((n, dm), jnp.float32)
    res = jnp.zeros((n, dm), jnp.float32)
    for k in range(keff):
        e = jnp.exp(logit_s[:, k * dm:(k + 1) * dm] - m)
        ssum = ssum + e
        res = res + e * wsrc_s[:, k * dm:(k + 1) * dm]
    res = res / ssum
    o_ref[0] = _dot(res, fc2w[...]) + fc2b[...] + feats


def _td_kernel(nc_ref, c_ref, f_ref, w, b, o_ref):
    newc = nc_ref[0]
    xyz = c_ref[0]
    feats = f_ref[0]
    m, n = newc.shape[0], xyz.shape[0]

    dists = _sqdist(newc, xyz)
    src = jnp.concatenate([xyz, feats], axis=1)
    col = jax.lax.broadcasted_iota(jnp.int32, (m, n), 1)
    acc = None
    for k in range(_K):
        idx = jnp.argmin(dists, axis=-1)
        sel = col == idx[:, None]
        dists = jnp.where(sel, jnp.inf, dists)
        g = jnp.dot(sel.astype(jnp.float32), src, precision=_HIGH)
        gx = g[:, 0:3] - newc
        gf = g[:, 3:]
        val = _relu(_dot(jnp.concatenate([gx, gf], axis=1), w[...]) + b[...])
        acc = val if acc is None else jnp.maximum(acc, val)
    o_ref[0] = acc


def _fps_kernel(npoint, c_ref, o_ref):
    cx = c_ref[:, 0, :]
    cy = c_ref[:, 1, :]
    cz = c_ref[:, 2, :]
    bsz, n = cx.shape
    lane = jax.lax.broadcasted_iota(jnp.int32, (bsz, n), 1)
    mlane = jax.lax.broadcasted_iota(jnp.int32, (bsz, npoint), 1)

    def body(i, state):
        dist, far, ox, oy, oz = state
        sel = (lane == far[:, None]).astype(jnp.float32)
        fx = jnp.sum(sel * cx, -1)
        fy = jnp.sum(sel * cy, -1)
        fz = jnp.sum(sel * cz, -1)
        put = mlane == i
        ox = jnp.where(put, fx[:, None], ox)
        oy = jnp.where(put, fy[:, None], oy)
        oz = jnp.where(put, fz[:, None], oz)
        d = (cx - fx[:, None]) ** 2 + (cy - fy[:, None]) ** 2 \
            + (cz - fz[:, None]) ** 2
        dist = jnp.minimum(dist, d)
        far = jnp.argmax(dist, axis=-1).astype(jnp.int32)
        return dist, far, ox, oy, oz

    zc = jnp.zeros((bsz, npoint), jnp.float32)
    _, _, ox, oy, oz = jax.lax.fori_loop(
        0, npoint, body,
        (jnp.full((bsz, n), 1e10, jnp.float32),
         jnp.zeros((bsz,), jnp.int32), zc, zc, zc))
    o_ref[:, 0, :] = ox
    o_ref[:, 1, :] = oy
    o_ref[:, 2, :] = oz


def _tail_kernel(f_ref, w0, b0, w1, b1, w2, b2, o_ref):
    f = f_ref[...]
    g = jnp.mean(f, axis=1)
    g = _relu(_dot(g, w0[...]) + b0[...])
    g = _relu(_dot(g, w1[...]) + b1[...])
    o_ref[...] = _dot(g, w2[...]) + b2[...]


def _per_batch(fn, batch_args, weight_args, out_row, scratch_shapes=()):
    bsz = batch_args[0].shape[0]
    in_specs = []
    for a in batch_args:
        in_specs.append(pl.BlockSpec(
            (1,) + a.shape[1:], lambda b, _n=a.ndim: (b,) + (0,) * (_n - 1)))
    for w in weight_args:
        in_specs.append(pl.BlockSpec(
            w.shape, lambda b, _n=w.ndim: (0,) * _n))
    out_spec = pl.BlockSpec(
        (1,) + out_row, lambda b, _n=len(out_row): (b,) + (0,) * _n)
    return pl.pallas_call(
        fn,
        grid=(bsz,),
        in_specs=in_specs,
        out_specs=out_spec,
        out_shape=jax.ShapeDtypeStruct((bsz,) + out_row, jnp.float32),
        scratch_shapes=list(scratch_shapes),
        compiler_params=pltpu.CompilerParams(
            dimension_semantics=("parallel",)),
    )(*batch_args, *weight_args)


def _lw(lin):
    return lin["W"], lin["b"][None, :]


def _transformer(tp, coords, feats):
    bsz, n, _ = coords.shape
    d = feats.shape[2]
    dm = tp["w_qs"]["W"].shape[1]
    keff = min(_K, n)
    ws = []
    for name in ("fc1", "fc2", "w_qs", "w_ks", "w_vs",
                 "delta1", "delta2", "gamma1", "gamma2"):
        w, b = _lw(tp[name])
        ws.extend([w, b])
    scratch = [pltpu.VMEM((n, keff * dm), jnp.float32),
               pltpu.VMEM((n, keff * dm), jnp.float32)]
    fn = functools.partial(_transformer_kernel, keff)
    return _per_batch(fn, [coords, feats], ws, (n, d), scratch)


def _fps(coords, npoint):
    bsz = coords.shape[0]
    fn = functools.partial(_fps_kernel, npoint)
    out = pl.pallas_call(
        fn, out_shape=jax.ShapeDtypeStruct((bsz, 3, npoint), jnp.float32),
    )(jnp.swapaxes(coords, 1, 2))
    return jnp.swapaxes(out, 1, 2)


def kernel(x, params):
    bsz, n, _ = x.shape
    coords = x[..., :3]
    p = params

    w0, b0 = _lw(p["mlp_first"][0])
    w1, b1 = _lw(p["mlp_first"][1])
    f = _per_batch(_mlp_first_kernel, [x], [w0, b0, w1, b1],
                   (n, w1.shape[1]))

    f = _transformer(p["transformer1"], coords, f)

    n_block = len(p["td"])
    for i in range(n_block):
        m = n // 4 ** (i + 1)
        newc = _fps(coords, m)
        wtd, btd = _lw(p["td"][i])
        f = _per_batch(_td_kernel, [newc, coords, f], [wtd, btd],
                       (m, wtd.shape[1]))
        coords = newc
        f = _transformer(p["pt"][i], coords, f)

    wl0, bl0 = _lw(p["mlp_last"][0])
    wl1, bl1 = _lw(p["mlp_last"][1])
    wl2, bl2 = _lw(p["mlp_last"][2])
    out = pl.pallas_call(
        _tail_kernel,
        out_shape=jax.ShapeDtypeStruct((bsz, wl2.shape[1]), jnp.float32),
    )(f, wl0, bl0, wl1, bl1, wl2, bl2)
    return out


# speedup vs baseline: 9.6707x; 1.4089x over previous
import functools

import jax
import jax.numpy as jnp
from jax.experimental import pallas as pl
from jax.experimental.pallas import tpu as pltpu

_HIGH = jax.lax.Precision.HIGHEST
_K = 16


def _dot(a, b):
    return jnp.dot(a, b)


def _relu(v):
    return jnp.maximum(v, 0.0)


def _sqdist(a, b):
    d = -2.0 * jax.lax.dot_general(a, b, (((1,), (1,)), ((), ())))
    d = d + jnp.sum(a * a, -1)[:, None]
    d = d + jnp.sum(b * b, -1)[None, :]
    return d


def _mlp_first_kernel(x_ref, w0, b0, w1, b1, o_ref):
    x = x_ref[0]
    h = _relu(_dot(x, w0[...]) + b0[...])
    o_ref[0] = _dot(h, w1[...]) + b1[...]


def _transformer_kernel(keff, xyz_ref, f_ref,
                        fc1w, fc1b, fc2w, fc2b,
                        qw, qb, kw, kb, vw, vb,
                        bd1w, bd1b, bd2w, bd2b,
                        bg1w, bg1b, bg2w, bg2b,
                        o_ref, kf_s, vf_s, x_s):
    xyz = xyz_ref[0]
    feats = f_ref[0]
    n = xyz.shape[0]
    dm = qw.shape[1]

    dists = _sqdist(xyz, xyz)
    h = _dot(feats, fc1w[...]) + fc1b[...]
    q = _dot(h, qw[...]) + qb[...]
    kf = _dot(h, kw[...]) + kb[...]
    vf = _dot(h, vw[...]) + vb[...]
    src = jnp.concatenate([kf, vf, xyz], axis=1)

    col = jax.lax.broadcasted_iota(jnp.int32, (n, n), 1)
    scale = 1.0 / (dm ** 0.5)
    for k in range(keff):
        idx = jnp.argmin(dists, axis=-1)
        sel = col == idx[:, None]
        dists = jnp.where(sel, jnp.inf, dists)
        g = jnp.dot(sel.astype(jnp.float32), src, precision=_HIGH)
        kf_s[:, k * dm:(k + 1) * dm] = g[:, 0:dm]
        vf_s[:, k * dm:(k + 1) * dm] = g[:, dm:2 * dm]
        x_s[:, k * 3:(k + 1) * 3] = g[:, 2 * dm:2 * dm + 3]

    pos = jnp.concatenate([xyz] * keff, axis=1) - x_s[...]
    pe = _dot(_relu(_dot(pos, bd1w[...]) + bd1b[...]),
              bd2w[...]) + bd2b[...]
    att = jnp.concatenate([q] * keff, axis=1) - kf_s[...] + pe
    lg = (_dot(_relu(_dot(att, bg1w[...]) + bg1b[...]),
               bg2w[...]) + bg2b[...]) * scale
    ws = vf_s[...] + pe

    m = lg[:, 0:dm]
    for k in range(1, keff):
        m = jnp.maximum(m, lg[:, k * dm:(k + 1) * dm])
    ssum = jnp.zeros((n, dm), jnp.float32)
    res = jnp.zeros((n, dm), jnp.float32)
    for k in range(keff):
        e = jnp.exp(lg[:, k * dm:(k + 1) * dm] - m)
        ssum = ssum + e
        res = res + e * ws[:, k * dm:(k + 1) * dm]
    res = res / ssum
    o_ref[0] = _dot(res, fc2w[...]) + fc2b[...] + feats


def _td_kernel(nc_ref, c_ref, f_ref, w, b, o_ref):
    newc = nc_ref[0]
    xyz = c_ref[0]
    feats = f_ref[0]
    m, n = newc.shape[0], xyz.shape[0]

    dists = _sqdist(newc, xyz)
    src = jnp.concatenate([xyz, feats], axis=1)
    col = jax.lax.broadcasted_iota(jnp.int32, (m, n), 1)
    acc = None
    for k in range(_K):
        idx = jnp.argmin(dists, axis=-1)
        sel = col == idx[:, None]
        dists = jnp.where(sel, jnp.inf, dists)
        g = jnp.dot(sel.astype(jnp.float32), src, precision=_HIGH)
        gx = g[:, 0:3] - newc
        gf = g[:, 3:]
        val = _relu(_dot(jnp.concatenate([gx, gf], axis=1), w[...]) + b[...])
        acc = val if acc is None else jnp.maximum(acc, val)
    o_ref[0] = acc


def _fps_kernel(npoint, c_ref, o_ref):
    cx = c_ref[:, 0, :]
    cy = c_ref[:, 1, :]
    cz = c_ref[:, 2, :]
    bsz, n = cx.shape
    lane = jax.lax.broadcasted_iota(jnp.int32, (bsz, n), 1)
    mlane = jax.lax.broadcasted_iota(jnp.int32, (bsz, npoint), 1)

    def body(i, state):
        dist, far, ox, oy, oz = state
        sel = (lane == far[:, None]).astype(jnp.float32)
        fx = jnp.sum(sel * cx, -1)
        fy = jnp.sum(sel * cy, -1)
        fz = jnp.sum(sel * cz, -1)
        put = mlane == i
        ox = jnp.where(put, fx[:, None], ox)
        oy = jnp.where(put, fy[:, None], oy)
        oz = jnp.where(put, fz[:, None], oz)
        d = (cx - fx[:, None]) ** 2 + (cy - fy[:, None]) ** 2 \
            + (cz - fz[:, None]) ** 2
        dist = jnp.minimum(dist, d)
        far = jnp.argmax(dist, axis=-1).astype(jnp.int32)
        return dist, far, ox, oy, oz

    zc = jnp.zeros((bsz, npoint), jnp.float32)
    _, _, ox, oy, oz = jax.lax.fori_loop(
        0, npoint, body,
        (jnp.full((bsz, n), 1e10, jnp.float32),
         jnp.zeros((bsz,), jnp.int32), zc, zc, zc))
    o_ref[:, 0, :] = ox
    o_ref[:, 1, :] = oy
    o_ref[:, 2, :] = oz


def _tail_kernel(f_ref, w0, b0, w1, b1, w2, b2, o_ref):
    f = f_ref[...]
    g = jnp.mean(f, axis=1)
    g = _relu(_dot(g, w0[...]) + b0[...])
    g = _relu(_dot(g, w1[...]) + b1[...])
    o_ref[...] = _dot(g, w2[...]) + b2[...]


def _per_batch(fn, batch_args, weight_args, out_row, scratch_shapes=()):
    bsz = batch_args[0].shape[0]
    in_specs = []
    for a in batch_args:
        in_specs.append(pl.BlockSpec(
            (1,) + a.shape[1:], lambda b, _n=a.ndim: (b,) + (0,) * (_n - 1)))
    for w in weight_args:
        in_specs.append(pl.BlockSpec(
            w.shape, lambda b, _n=w.ndim: (0,) * _n))
    out_spec = pl.BlockSpec(
        (1,) + out_row, lambda b, _n=len(out_row): (b,) + (0,) * _n)
    return pl.pallas_call(
        fn,
        grid=(bsz,),
        in_specs=in_specs,
        out_specs=out_spec,
        out_shape=jax.ShapeDtypeStruct((bsz,) + out_row, jnp.float32),
        scratch_shapes=list(scratch_shapes),
        compiler_params=pltpu.CompilerParams(
            dimension_semantics=("parallel",)),
    )(*batch_args, *weight_args)


def _lw(lin):
    return lin["W"], lin["b"][None, :]


def _transformer(tp, coords, feats):
    bsz, n, _ = coords.shape
    d = feats.shape[2]
    dm = tp["w_qs"]["W"].shape[1]
    keff = min(_K, n)
    eye = jnp.eye(keff, dtype=jnp.float32)
    ws = []
    for name in ("fc1", "fc2", "w_qs", "w_ks", "w_vs"):
        w, b = _lw(tp[name])
        ws.extend([w, b])
    for name in ("delta1", "delta2", "gamma1", "gamma2"):
        w, b = _lw(tp[name])
        ws.extend([jnp.kron(eye, w), jnp.tile(b, (1, keff))])
    scratch = [pltpu.VMEM((n, keff * dm), jnp.float32),
               pltpu.VMEM((n, keff * dm), jnp.float32),
               pltpu.VMEM((n, keff * 3), jnp.float32)]
    fn = functools.partial(_transformer_kernel, keff)
    return _per_batch(fn, [coords, feats], ws, (n, d), scratch)


def _fps(coords, npoint):
    bsz = coords.shape[0]
    fn = functools.partial(_fps_kernel, npoint)
    out = pl.pallas_call(
        fn, out_shape=jax.ShapeDtypeStruct((bsz, 3, npoint), jnp.float32),
    )(jnp.swapaxes(coords, 1, 2))
    return jnp.swapaxes(out, 1, 2)


def kernel(x, params):
    bsz, n, _ = x.shape
    coords = x[..., :3]
    p = params

    w0, b0 = _lw(p["mlp_first"][0])
    w1, b1 = _lw(p["mlp_first"][1])
    f = _per_batch(_mlp_first_kernel, [x], [w0, b0, w1, b1],
                   (n, w1.shape[1]))

    f = _transformer(p["transformer1"], coords, f)

    n_block = len(p["td"])
    for i in range(n_block):
        m = n // 4 ** (i + 1)
        newc = _fps(coords, m)
        wtd, btd = _lw(p["td"][i])
        f = _per_batch(_td_kernel, [newc, coords, f], [wtd, btd],
                       (m, wtd.shape[1]))
        coords = newc
        f = _transformer(p["pt"][i], coords, f)

    wl0, bl0 = _lw(p["mlp_last"][0])
    wl1, bl1 = _lw(p["mlp_last"][1])
    wl2, bl2 = _lw(p["mlp_last"][2])
    out = pl.pallas_call(
        _tail_kernel,
        out_shape=jax.ShapeDtypeStruct((bsz, wl2.shape[1]), jnp.float32),
    )(f, wl0, bl0, wl1, bl1, wl2, bl2)
    return out


# speedup vs baseline: 13.9296x; 1.4404x over previous
import functools

import jax
import jax.numpy as jnp
from jax.experimental import pallas as pl
from jax.experimental.pallas import tpu as pltpu

_HIGH = jax.lax.Precision.HIGHEST
_K = 16


def _dot(a, b):
    return jnp.dot(a, b)


def _relu(v):
    return jnp.maximum(v, 0.0)


def _sqdist(a, b):
    d = -2.0 * jax.lax.dot_general(a, b, (((1,), (1,)), ((), ())))
    d = d + jnp.sum(a * a, -1)[:, None]
    d = d + jnp.sum(b * b, -1)[None, :]
    return d


def _oh_gather(sel, srch, srcl):
    oh = sel.astype(jnp.bfloat16)
    dims = (((1,), (0,)), ((), ()))
    return (jax.lax.dot_general(oh, srch, dims,
                                preferred_element_type=jnp.float32)
            + jax.lax.dot_general(oh, srcl, dims,
                                  preferred_element_type=jnp.float32))


def _mlp_first_kernel(x_ref, w0, b0, w1, b1, o_ref):
    x = x_ref[0]
    h = _relu(_dot(x, w0[...]) + b0[...])
    o_ref[0] = _dot(h, w1[...]) + b1[...]


def _transformer_kernel(keff, xyz_ref, f_ref,
                        fc1w, fc1b, fc2w, fc2b,
                        qw, qb, kw, kb, vw, vb,
                        bd1w, bd1b, bd2w, bd2b,
                        bg1w, bg1b, bg2w, bg2b,
                        o_ref, kf_s, vf_s, x_s):
    xyz = xyz_ref[0]
    feats = f_ref[0]
    n = xyz.shape[0]
    dm = qw.shape[1]

    dists = _sqdist(xyz, xyz)
    h = _dot(feats, fc1w[...]) + fc1b[...]
    q = _dot(h, qw[...]) + qb[...]
    kf = _dot(h, kw[...]) + kb[...]
    vf = _dot(h, vw[...]) + vb[...]
    src = jnp.concatenate([kf, vf, xyz], axis=1)

    col = jax.lax.broadcasted_iota(jnp.int32, (n, n), 1)
    scale = 1.0 / (dm ** 0.5)
    srch = src.astype(jnp.bfloat16)
    srcl = (src - srch.astype(jnp.float32)).astype(jnp.bfloat16)
    for k in range(keff):
        idx = jnp.argmin(dists, axis=-1)
        sel = col == idx[:, None]
        dists = jnp.where(sel, jnp.inf, dists)
        g = _oh_gather(sel, srch, srcl)
        kf_s[:, k * dm:(k + 1) * dm] = g[:, 0:dm]
        vf_s[:, k * dm:(k + 1) * dm] = g[:, dm:2 * dm]
        x_s[:, k * 3:(k + 1) * 3] = g[:, 2 * dm:2 * dm + 3]

    pos = jnp.concatenate([xyz] * keff, axis=1) - x_s[...]
    pe = _dot(_relu(_dot(pos, bd1w[...]) + bd1b[...]),
              bd2w[...]) + bd2b[...]
    att = jnp.concatenate([q] * keff, axis=1) - kf_s[...] + pe
    lg = (_dot(_relu(_dot(att, bg1w[...]) + bg1b[...]),
               bg2w[...]) + bg2b[...]) * scale
    ws = vf_s[...] + pe

    m = lg[:, 0:dm]
    for k in range(1, keff):
        m = jnp.maximum(m, lg[:, k * dm:(k + 1) * dm])
    ssum = jnp.zeros((n, dm), jnp.float32)
    res = jnp.zeros((n, dm), jnp.float32)
    for k in range(keff):
        e = jnp.exp(lg[:, k * dm:(k + 1) * dm] - m)
        ssum = ssum + e
        res = res + e * ws[:, k * dm:(k + 1) * dm]
    res = res / ssum
    o_ref[0] = _dot(res, fc2w[...]) + fc2b[...] + feats


def _td_kernel(nc_ref, c_ref, f_ref, w, b, o_ref):
    newc = nc_ref[0]
    xyz = c_ref[0]
    feats = f_ref[0]
    m, n = newc.shape[0], xyz.shape[0]

    dists = _sqdist(newc, xyz)
    src = jnp.concatenate([xyz, feats], axis=1)
    col = jax.lax.broadcasted_iota(jnp.int32, (m, n), 1)
    srch = src.astype(jnp.bfloat16)
    srcl = (src - srch.astype(jnp.float32)).astype(jnp.bfloat16)
    acc = None
    for k in range(_K):
        idx = jnp.argmin(dists, axis=-1)
        sel = col == idx[:, None]
        dists = jnp.where(sel, jnp.inf, dists)
        g = _oh_gather(sel, srch, srcl)
        gx = g[:, 0:3] - newc
        gf = g[:, 3:]
        val = _relu(_dot(jnp.concatenate([gx, gf], axis=1), w[...]) + b[...])
        acc = val if acc is None else jnp.maximum(acc, val)
    o_ref[0] = acc


def _fps_kernel(npoint, c_ref, o_ref):
    cx = c_ref[:, 0, :]
    cy = c_ref[:, 1, :]
    cz = c_ref[:, 2, :]
    bsz, n = cx.shape
    lane = jax.lax.broadcasted_iota(jnp.int32, (bsz, n), 1)
    mlane = jax.lax.broadcasted_iota(jnp.int32, (bsz, npoint), 1)

    def body(i, state):
        dist, far, ox, oy, oz = state
        sel = (lane == far[:, None]).astype(jnp.float32)
        fx = jnp.sum(sel * cx, -1)
        fy = jnp.sum(sel * cy, -1)
        fz = jnp.sum(sel * cz, -1)
        put = mlane == i
        ox = jnp.where(put, fx[:, None], ox)
        oy = jnp.where(put, fy[:, None], oy)
        oz = jnp.where(put, fz[:, None], oz)
        d = (cx - fx[:, None]) ** 2 + (cy - fy[:, None]) ** 2 \
            + (cz - fz[:, None]) ** 2
        dist = jnp.minimum(dist, d)
        far = jnp.argmax(dist, axis=-1).astype(jnp.int32)
        return dist, far, ox, oy, oz

    zc = jnp.zeros((bsz, npoint), jnp.float32)
    _, _, ox, oy, oz = jax.lax.fori_loop(
        0, npoint, body,
        (jnp.full((bsz, n), 1e10, jnp.float32),
         jnp.zeros((bsz,), jnp.int32), zc, zc, zc))
    o_ref[:, 0, :] = ox
    o_ref[:, 1, :] = oy
    o_ref[:, 2, :] = oz


def _tail_kernel(f_ref, w0, b0, w1, b1, w2, b2, o_ref):
    f = f_ref[...]
    g = jnp.mean(f, axis=1)
    g = _relu(_dot(g, w0[...]) + b0[...])
    g = _relu(_dot(g, w1[...]) + b1[...])
    o_ref[...] = _dot(g, w2[...]) + b2[...]


def _per_batch(fn, batch_args, weight_args, out_row, scratch_shapes=()):
    bsz = batch_args[0].shape[0]
    in_specs = []
    for a in batch_args:
        in_specs.append(pl.BlockSpec(
            (1,) + a.shape[1:], lambda b, _n=a.ndim: (b,) + (0,) * (_n - 1)))
    for w in weight_args:
        in_specs.append(pl.BlockSpec(
            w.shape, lambda b, _n=w.ndim: (0,) * _n))
    out_spec = pl.BlockSpec(
        (1,) + out_row, lambda b, _n=len(out_row): (b,) + (0,) * _n)
    return pl.pallas_call(
        fn,
        grid=(bsz,),
        in_specs=in_specs,
        out_specs=out_spec,
        out_shape=jax.ShapeDtypeStruct((bsz,) + out_row, jnp.float32),
        scratch_shapes=list(scratch_shapes),
        compiler_params=pltpu.CompilerParams(
            dimension_semantics=("parallel",)),
    )(*batch_args, *weight_args)


def _lw(lin):
    return lin["W"], lin["b"][None, :]


def _transformer(tp, coords, feats):
    bsz, n, _ = coords.shape
    d = feats.shape[2]
    dm = tp["w_qs"]["W"].shape[1]
    keff = min(_K, n)
    eye = jnp.eye(keff, dtype=jnp.float32)
    ws = []
    for name in ("fc1", "fc2", "w_qs", "w_ks", "w_vs"):
        w, b = _lw(tp[name])
        ws.extend([w, b])
    for name in ("delta1", "delta2", "gamma1", "gamma2"):
        w, b = _lw(tp[name])
        ws.extend([jnp.kron(eye, w), jnp.tile(b, (1, keff))])
    scratch = [pltpu.VMEM((n, keff * dm), jnp.float32),
               pltpu.VMEM((n, keff * dm), jnp.float32),
               pltpu.VMEM((n, keff * 3), jnp.float32)]
    fn = functools.partial(_transformer_kernel, keff)
    return _per_batch(fn, [coords, feats], ws, (n, d), scratch)


def _fps(coords, npoint):
    bsz = coords.shape[0]
    fn = functools.partial(_fps_kernel, npoint)
    out = pl.pallas_call(
        fn, out_shape=jax.ShapeDtypeStruct((bsz, 3, npoint), jnp.float32),
    )(jnp.swapaxes(coords, 1, 2))
    return jnp.swapaxes(out, 1, 2)


def kernel(x, params):
    bsz, n, _ = x.shape
    coords = x[..., :3]
    p = params

    w0, b0 = _lw(p["mlp_first"][0])
    w1, b1 = _lw(p["mlp_first"][1])
    f = _per_batch(_mlp_first_kernel, [x], [w0, b0, w1, b1],
                   (n, w1.shape[1]))

    f = _transformer(p["transformer1"], coords, f)

    n_block = len(p["td"])
    for i in range(n_block):
        m = n // 4 ** (i + 1)
        newc = _fps(coords, m)
        wtd, btd = _lw(p["td"][i])
        f = _per_batch(_td_kernel, [newc, coords, f], [wtd, btd],
                       (m, wtd.shape[1]))
        coords = newc
        f = _transformer(p["pt"][i], coords, f)

    wl0, bl0 = _lw(p["mlp_last"][0])
    wl1, bl1 = _lw(p["mlp_last"][1])
    wl2, bl2 = _lw(p["mlp_last"][2])
    out = pl.pallas_call(
        _tail_kernel,
        out_shape=jax.ShapeDtypeStruct((bsz, wl2.shape[1]), jnp.float32),
    )(f, wl0, bl0, wl1, bl1, wl2, bl2)
    return out
